# Initial kernel scaffold; baseline (speedup 1.0000x reference)
import jax
import jax.numpy as jnp
from jax.experimental import pallas as pl


def kernel(queries, keys, values, attn_mask):
    raise NotImplementedError("write your pallas kernel here")


# speedup vs baseline: 9.9376x; 9.9376x over previous
import functools
import math

import numpy as np
import jax
import jax.numpy as jnp
from jax.experimental import pallas as pl
from jax.experimental.pallas import tpu as pltpu

_L = 4096
_R = 64
_TOPK = 8
_FB = 128
_PAD = 128


def _dft_consts():
    r = _R
    n = np.arange(r)
    ang = 2.0 * np.pi * np.outer(n, n) / r
    c, s = np.cos(ang), -np.sin(ang)
    cs = np.concatenate([c, s], axis=0)
    cst = np.concatenate([c.T, s.T], axis=1)
    ang2 = 2.0 * np.pi * np.outer(n, n) / _L
    tr = np.cos(ang2).reshape(r, 1, r)
    ti = (-np.sin(ang2)).reshape(r, 1, r)
    e1r, e1i = np.cos(ang), np.sin(ang)
    pr = np.cos(ang2)
    pi_ = np.sin(ang2)
    f32 = lambda a: jnp.asarray(a, dtype=jnp.float32)
    return (f32(cs), f32(cst), f32(tr), f32(ti),
            f32(e1r), f32(e1i), f32(pr), f32(pi_))


def _fft_xspec_kernel(cs_ref, cst_ref, tr_ref, ti_ref, q_ref, k_ref,
                      sr_ref, si_ref):
    fi = pl.program_id(1)
    cs = cs_ref[...]
    cst = cst_ref[...]
    tr = tr_ref[...]
    ti = ti_ref[...]

    def fwd_fft(x):
        x3 = x.reshape(_R, _R, _FB)
        x3t = jnp.transpose(x3, (0, 2, 1))
        xm = x3t.reshape(_R, _FB * _R)
        y = jnp.dot(cs, xm, preferred_element_type=jnp.float32,
                    precision=jax.lax.Precision.HIGHEST)
        yr = y[:_R].reshape(_R, _FB, _R)
        yi = y[_R:].reshape(_R, _FB, _R)
        y2r = yr * tr - yi * ti
        y2i = yr * ti + yi * tr
        a = jnp.dot(y2r.reshape(_R * _FB, _R), cst,
                    preferred_element_type=jnp.float32,
                    precision=jax.lax.Precision.HIGHEST)
        b = jnp.dot(y2i.reshape(_R * _FB, _R), cst,
                    preferred_element_type=jnp.float32,
                    precision=jax.lax.Precision.HIGHEST)
        zr = a[:, :_R] - b[:, _R:]
        zi = b[:, :_R] + a[:, _R:]
        return zr, zi

    qr, qi = fwd_fft(q_ref[0])
    kr, ki = fwd_fft(k_ref[0])
    pr = jnp.sum((qr * kr + qi * ki).reshape(_R, _FB, _R), axis=1)
    pi_ = jnp.sum((qi * kr - qr * ki).reshape(_R, _FB, _R), axis=1)

    @pl.when(fi == 0)
    def _():
        sr_ref[0] = pr
        si_ref[0] = pi_

    @pl.when(fi != 0)
    def _():
        sr_ref[0] += pr
        si_ref[0] += pi_


def _delays_kernel(e1r_ref, e1i_ref, pr_ref, pi_ref, sr_ref, si_ref,
                   w_ref, d_ref):
    sr = sr_ref[0]
    si = si_ref[0]
    e1r, e1i = e1r_ref[...], e1i_ref[...]
    pr, pi_ = pr_ref[...], pi_ref[...]
    ur = (jnp.dot(sr, e1r, preferred_element_type=jnp.float32,
                    precision=jax.lax.Precision.HIGHEST)
          - jnp.dot(si, e1i, preferred_element_type=jnp.float32,
                    precision=jax.lax.Precision.HIGHEST))
    ui = (jnp.dot(sr, e1i, preferred_element_type=jnp.float32,
                    precision=jax.lax.Precision.HIGHEST)
          + jnp.dot(si, e1r, preferred_element_type=jnp.float32,
                    precision=jax.lax.Precision.HIGHEST))
    vr = ur * pr - ui * pi_
    vi = ur * pi_ + ui * pr
    mr = (jnp.dot(e1r, vr, preferred_element_type=jnp.float32,
                    precision=jax.lax.Precision.HIGHEST)
          - jnp.dot(e1i, vi, preferred_element_type=jnp.float32,
                    precision=jax.lax.Precision.HIGHEST))
    mv = mr * jnp.float32(1.0 / (_L * 1024.0))
    tau = (jax.lax.broadcasted_iota(jnp.int32, (_R, _R), 0) * _R
           + jax.lax.broadcasted_iota(jnp.int32, (_R, _R), 1))
    vals = []
    idxs = []
    for _ in range(_TOPK):
        m = jnp.max(mv)
        sel = jnp.min(jnp.where(mv == m, tau, jnp.int32(2 ** 30)))
        vals.append(m)
        idxs.append(sel)
        mv = jnp.where(tau == sel, jnp.float32(-3.0e38), mv)
    exps = [jnp.exp(v - vals[0]) for v in vals]
    tot = functools.reduce(lambda a, b: a + b, exps)
    lane = jax.lax.broadcasted_iota(jnp.int32, (1, _PAD), 1)
    wvec = jnp.zeros((1, _PAD), jnp.float32)
    dvec = jnp.zeros((1, _PAD), jnp.int32)
    for i in range(_TOPK):
        wvec = jnp.where(lane == i, exps[i] / tot, wvec)
        dvec = jnp.where(lane == i, idxs[i], dvec)
    w_ref[...] = wvec.reshape(1, 1, _PAD)
    d_ref[...] = dvec.reshape(1, 1, _PAD)


def _agg_kernel(w_ref, d_ref, v_ref, o_ref, v2_ref):
    b = pl.program_id(0)
    v = v_ref[0]
    v2_ref[0:_L, :] = v
    v2_ref[_L:2 * _L, :] = v
    d0 = d_ref[b, 0, 0]
    o_ref[0] = w_ref[b, 0, 0] * v2_ref[pl.ds(d0, _L), :]
    for i in range(1, _TOPK):
        di = d_ref[b, 0, i]
        o_ref[0] += w_ref[b, 0, i] * v2_ref[pl.ds(di, _L), :]


def kernel(queries, keys, values, attn_mask):
    del attn_mask
    b, l, h, e = queries.shape
    f = h * e
    nf = f // _FB
    cs, cst, tr, ti, e1r, e1i, pr, pi_ = _dft_consts()

    q2 = queries.reshape(b, l, f)
    k2 = keys.reshape(b, l, f)

    const_spec = lambda shp: pl.BlockSpec(shp, lambda bi, fi: (0,) * len(shp))
    sr, si = pl.pallas_call(
        _fft_xspec_kernel,
        grid=(b, nf),
        in_specs=[
            const_spec((2 * _R, _R)),
            const_spec((_R, 2 * _R)),
            const_spec((_R, 1, _R)),
            const_spec((_R, 1, _R)),
            pl.BlockSpec((1, l, _FB), lambda bi, fi: (bi, 0, fi)),
            pl.BlockSpec((1, l, _FB), lambda bi, fi: (bi, 0, fi)),
        ],
        out_specs=[
            pl.BlockSpec((1, _R, _R), lambda bi, fi: (bi, 0, 0)),
            pl.BlockSpec((1, _R, _R), lambda bi, fi: (bi, 0, 0)),
        ],
        out_shape=[
            jax.ShapeDtypeStruct((b, _R, _R), jnp.float32),
            jax.ShapeDtypeStruct((b, _R, _R), jnp.float32),
        ],
    )(cs, cst, tr, ti, q2, k2)

    wpad, dpad = pl.pallas_call(
        _delays_kernel,
        grid=(b,),
        in_specs=[
            pl.BlockSpec((_R, _R), lambda bi: (0, 0)),
            pl.BlockSpec((_R, _R), lambda bi: (0, 0)),
            pl.BlockSpec((_R, _R), lambda bi: (0, 0)),
            pl.BlockSpec((_R, _R), lambda bi: (0, 0)),
            pl.BlockSpec((1, _R, _R), lambda bi: (bi, 0, 0)),
            pl.BlockSpec((1, _R, _R), lambda bi: (bi, 0, 0)),
        ],
        out_specs=[
            pl.BlockSpec((1, 1, _PAD), lambda bi: (bi, 0, 0)),
            pl.BlockSpec((1, 1, _PAD), lambda bi: (bi, 0, 0)),
        ],
        out_shape=[
            jax.ShapeDtypeStruct((b, 1, _PAD), jnp.float32),
            jax.ShapeDtypeStruct((b, 1, _PAD), jnp.int32),
        ],
    )(e1r, e1i, pr, pi_, sr, si)

    fa = 128
    v2 = values.reshape(b, l, f)
    out = pl.pallas_call(
        _agg_kernel,
        grid=(b, f // fa),
        in_specs=[
            pl.BlockSpec(memory_space=pltpu.SMEM),
            pl.BlockSpec(memory_space=pltpu.SMEM),
            pl.BlockSpec((1, l, fa), lambda bi, fi: (bi, 0, fi)),
        ],
        out_specs=pl.BlockSpec((1, l, fa), lambda bi, fi: (bi, 0, fi)),
        out_shape=jax.ShapeDtypeStruct((b, l, f), jnp.float32),
        scratch_shapes=[pltpu.VMEM((2 * _L, fa), jnp.float32)],
    )(wpad, dpad, v2)
    return out.reshape(b, l, h, e)


# speedup vs baseline: 15.3259x; 1.5422x over previous
import functools

import numpy as np
import jax
import jax.numpy as jnp
from jax import lax
from jax.experimental import pallas as pl
from jax.experimental.pallas import tpu as pltpu
from jax.experimental.pallas import tpu_sc as plsc

_L = 4096
_R = 64
_TOPK = 8
_FB = 128
_PAD = 128


def _dft_consts():
    r = _R
    n = np.arange(r)
    ang = 2.0 * np.pi * np.outer(n, n) / r
    c, s = np.cos(ang), -np.sin(ang)
    cs = np.concatenate([c, s], axis=0)
    cst = np.concatenate([c.T, s.T], axis=1)
    def split(a):
        hi = a.astype(np.float32).astype(jnp.bfloat16)
        lo = (a - np.asarray(hi, np.float32)).astype(jnp.bfloat16)
        return np.asarray(hi), np.asarray(lo)
    cs_h, cs_l = split(cs)
    csb = np.concatenate([cs_h, cs_h, cs_l], axis=1)
    cst_h, cst_l = split(cst)
    cstb = np.concatenate([cst_h, cst_h, cst_l], axis=0)
    cst2 = np.concatenate([-(s.T), c.T], axis=1)
    cst2_h, cst2_l = split(cst2)
    cstb2 = np.concatenate([cst2_h, cst2_h, cst2_l], axis=0)
    ang2 = 2.0 * np.pi * np.outer(n, n) / _L
    ter = np.repeat(np.cos(ang2), _FB, axis=1)
    tei = np.repeat(-np.sin(ang2), _FB, axis=1)
    tr = ter
    ti = tei
    e1r, e1i = np.cos(ang), np.sin(ang)
    pr = np.cos(ang2)
    pi_ = np.sin(ang2)
    f32 = lambda a: jnp.asarray(a, dtype=jnp.float32)
    bf16 = lambda a: jnp.asarray(a, dtype=jnp.bfloat16)
    return (bf16(csb), bf16(cstb), bf16(cstb2), f32(tr), f32(ti),
            f32(e1r), f32(e1i), f32(pr), f32(pi_))


def _fft_xspec_kernel(cs_ref, cst_ref, cst2_ref, tr_ref, ti_ref,
                      q_ref, k_ref, sr_ref, si_ref):
    fi = pl.program_id(1)
    cs = cs_ref[...]
    cst = cst_ref[...]
    cst2 = cst2_ref[...]
    tr = tr_ref[...]
    ti = ti_ref[...]

    def split_k(x, axis):
        hi = x.astype(jnp.bfloat16)
        lo = (x - hi.astype(jnp.float32)).astype(jnp.bfloat16)
        return jnp.concatenate([hi, lo, hi], axis=axis)

    def swap_halves(z):
        return jnp.concatenate([z[:, _R:], z[:, :_R]], axis=1)

    def fwd_fft(x):
        xm = x.reshape(_R, _R * _FB)
        y = jnp.dot(cs, split_k(xm, 0),
                    preferred_element_type=jnp.float32)
        yr = y[:_R]
        yi = y[_R:]
        y2r = yr * tr - yi * ti
        y2i = yr * ti + yi * tr
        ar = jnp.transpose(y2r.reshape(_R, _R, _FB), (0, 2, 1))
        ai = jnp.transpose(y2i.reshape(_R, _R, _FB), (0, 2, 1))
        z = (jnp.dot(split_k(ar.reshape(_R * _FB, _R), 1), cst,
                     preferred_element_type=jnp.float32)
             + jnp.dot(split_k(ai.reshape(_R * _FB, _R), 1), cst2,
                       preferred_element_type=jnp.float32))
        return z

    zq = fwd_fft(q_ref[0])
    zk = fwd_fft(k_ref[0])
    p1 = zq * zk
    p2 = swap_halves(zq) * zk
    fsum = p1 + swap_halves(p1)
    gdif = p2 - swap_halves(p2)
    lane2 = jax.lax.broadcasted_iota(jnp.int32, (_R * _FB, 2 * _R), 1)
    s = jnp.where(lane2 < _R, fsum, -gdif)
    s3 = jnp.sum(s.reshape(_R, _FB, 2 * _R), axis=1)
    pr = s3[:, :_R]
    pi_ = s3[:, _R:]

    @pl.when(fi == 0)
    def _():
        sr_ref[0] = pr
        si_ref[0] = pi_

    @pl.when(fi != 0)
    def _():
        sr_ref[0] += pr
        si_ref[0] += pi_


def _delays_kernel(e1r_ref, e1i_ref, pr_ref, pi_ref, sr_ref, si_ref,
                   w_ref, d_ref, w16_ref, d16_ref):
    sr = sr_ref[0]
    si = si_ref[0]
    e1r, e1i = e1r_ref[...], e1i_ref[...]
    pr, pi_ = pr_ref[...], pi_ref[...]
    ur = (jnp.dot(sr, e1r, preferred_element_type=jnp.float32,
                    precision=jax.lax.Precision.HIGHEST)
          - jnp.dot(si, e1i, preferred_element_type=jnp.float32,
                    precision=jax.lax.Precision.HIGHEST))
    ui = (jnp.dot(sr, e1i, preferred_element_type=jnp.float32,
                    precision=jax.lax.Precision.HIGHEST)
          + jnp.dot(si, e1r, preferred_element_type=jnp.float32,
                    precision=jax.lax.Precision.HIGHEST))
    vr = ur * pr - ui * pi_
    vi = ur * pi_ + ui * pr
    mr = (jnp.dot(e1r, vr, preferred_element_type=jnp.float32,
                    precision=jax.lax.Precision.HIGHEST)
          - jnp.dot(e1i, vi, preferred_element_type=jnp.float32,
                    precision=jax.lax.Precision.HIGHEST))
    mv = mr * jnp.float32(1.0 / (_L * 1024.0))
    tau = (jax.lax.broadcasted_iota(jnp.int32, (_R, _R), 0) * _R
           + jax.lax.broadcasted_iota(jnp.int32, (_R, _R), 1))
    vals = []
    idxs = []
    for _ in range(_TOPK):
        m = jnp.max(mv)
        sel = jnp.min(jnp.where(mv == m, tau, jnp.int32(2 ** 30)))
        vals.append(m)
        idxs.append(sel)
        mv = jnp.where(tau == sel, jnp.float32(-3.0e38), mv)
    exps = [jnp.exp(v - vals[0]) for v in vals]
    tot = functools.reduce(lambda a, b: a + b, exps)
    row8 = jax.lax.broadcasted_iota(jnp.int32, (_TOPK, 16), 0)
    w16 = jnp.zeros((_TOPK, 16), jnp.float32)
    d16 = jnp.zeros((_TOPK, 16), jnp.int32)
    for i in range(_TOPK):
        w16 = jnp.where(row8 == i, exps[i] / tot, w16)
        d16 = jnp.where(row8 == i, idxs[i], d16)
    w16_ref[...] = w16.reshape(1, _TOPK, 16)
    d16_ref[...] = d16.reshape(1, _TOPK, 16)
    lane = jax.lax.broadcasted_iota(jnp.int32, (1, _PAD), 1)
    wvec = jnp.zeros((1, _PAD), jnp.float32)
    dvec = jnp.zeros((1, _PAD), jnp.int32)
    for i in range(_TOPK):
        wvec = jnp.where(lane == i, exps[i] / tot, wvec)
        dvec = jnp.where(lane == i, idxs[i], dvec)
    w_ref[...] = wvec.reshape(1, 1, _PAD)
    d_ref[...] = dvec.reshape(1, 1, _PAD)


_NC, _NS = 2, 16
_NW = _NC * _NS
_CH = 32


def _sc_aggregate(w16, d16, values):
    b, l, h, e = values.shape
    fdim = h * e
    v2d = values.reshape(b * l, fdim)
    rows_per_w = (b * l) // _NW
    mesh = plsc.VectorSubcoreMesh(core_axis_name="c", subcore_axis_name="s")

    @functools.partial(
        pl.kernel,
        out_type=jax.ShapeDtypeStruct((b * l, fdim), jnp.float32),
        mesh=mesh,
        scratch_types=[
            pltpu.VMEM((_CH,), jnp.int32),
            pltpu.VMEM((_CH, fdim), jnp.float32),
            pltpu.VMEM((_CH, fdim), jnp.float32),
            pltpu.VMEM((_TOPK, 16), jnp.float32),
            pltpu.VMEM((_TOPK, 16), jnp.int32),
            pltpu.SemaphoreType.DMA,
        ],
    )
    def k(w_hbm, d_hbm, v_hbm, out_hbm, idx_v, gat_v, acc_v, wv, dv, sem):
        wid = lax.axis_index("s") * _NC + lax.axis_index("c")
        row0 = wid * rows_per_w
        bb = row0 // _L
        base_b = bb * _L
        l0 = row0 - base_b
        pltpu.sync_copy(w_hbm.at[bb], wv)
        pltpu.sync_copy(d_hbm.at[bb], dv)
        lane = lax.iota(jnp.int32, 16)

        def subchunk(c, carry):
            lc = l0 + c * _CH
            for i in range(_TOPK):
                d_row = dv[i]
                w_row = wv[i]
                for half in range(_CH // 16):
                    t = lane + (lc + half * 16) + d_row
                    t = jnp.where(t >= _L, t - _L, t)
                    idx_v[pl.ds(half * 16, 16)] = t + base_b
                pltpu.async_copy(v_hbm.at[idx_v], gat_v, sem).wait()

                @plsc.parallel_loop(0, _CH * (fdim // 16), 1, unroll=8)
                def mac(j):
                    r = j // (fdim // 16)
                    s = (j % (fdim // 16)) * 16
                    g = gat_v[r, pl.ds(s, 16)]
                    if i == 0:
                        acc_v[r, pl.ds(s, 16)] = w_row * g
                    else:
                        acc_v[r, pl.ds(s, 16)] += w_row * g

            pltpu.sync_copy(acc_v, out_hbm.at[pl.ds(row0 + c * _CH, _CH)])
            return carry

        lax.fori_loop(0, rows_per_w // _CH, subchunk, 0)

    out = k(w16, d16, v2d)
    return out.reshape(b, l, h, e)


def _agg_kernel(w_ref, d_ref, v_ref, o_ref, v2_ref):
    b = pl.program_id(0)
    v = v_ref[0]
    v2_ref[0:_L, :] = v
    v2_ref[_L:2 * _L, :] = v
    d0 = d_ref[b, 0, 0]
    o_ref[0] = w_ref[b, 0, 0] * v2_ref[pl.ds(d0, _L), :]
    for i in range(1, _TOPK):
        di = d_ref[b, 0, i]
        o_ref[0] += w_ref[b, 0, i] * v2_ref[pl.ds(di, _L), :]


def kernel(queries, keys, values, attn_mask):
    del attn_mask
    b, l, h, e = queries.shape
    f = h * e
    nf = f // _FB
    cs, cst, cst2, tr, ti, e1r, e1i, pr, pi_ = _dft_consts()

    q2 = queries.reshape(b, l, f)
    k2 = keys.reshape(b, l, f)

    const_spec = lambda shp: pl.BlockSpec(shp, lambda bi, fi: (0,) * len(shp))
    sr, si = pl.pallas_call(
        _fft_xspec_kernel,
        grid=(b, nf),
        in_specs=[
            const_spec((2 * _R, 3 * _R)),
            const_spec((3 * _R, 2 * _R)),
            const_spec((3 * _R, 2 * _R)),
            const_spec((_R, _R * _FB)),
            const_spec((_R, _R * _FB)),
            pl.BlockSpec((1, l, _FB), lambda bi, fi: (bi, 0, fi)),
            pl.BlockSpec((1, l, _FB), lambda bi, fi: (bi, 0, fi)),
        ],
        out_specs=[
            pl.BlockSpec((1, _R, _R), lambda bi, fi: (bi, 0, 0)),
            pl.BlockSpec((1, _R, _R), lambda bi, fi: (bi, 0, 0)),
        ],
        out_shape=[
            jax.ShapeDtypeStruct((b, _R, _R), jnp.float32),
            jax.ShapeDtypeStruct((b, _R, _R), jnp.float32),
        ],
        compiler_params=pltpu.CompilerParams(
            dimension_semantics=("parallel", "arbitrary")),
    )(cs, cst, cst2, tr, ti, q2, k2)

    wpad, dpad, w16, d16 = pl.pallas_call(
        _delays_kernel,
        grid=(b,),
        in_specs=[
            pl.BlockSpec((_R, _R), lambda bi: (0, 0)),
            pl.BlockSpec((_R, _R), lambda bi: (0, 0)),
            pl.BlockSpec((_R, _R), lambda bi: (0, 0)),
            pl.BlockSpec((_R, _R), lambda bi: (0, 0)),
            pl.BlockSpec((1, _R, _R), lambda bi: (bi, 0, 0)),
            pl.BlockSpec((1, _R, _R), lambda bi: (bi, 0, 0)),
        ],
        out_specs=[
            pl.BlockSpec((1, 1, _PAD), lambda bi: (bi, 0, 0)),
            pl.BlockSpec((1, 1, _PAD), lambda bi: (bi, 0, 0)),
            pl.BlockSpec((1, _TOPK, 16), lambda bi: (bi, 0, 0)),
            pl.BlockSpec((1, _TOPK, 16), lambda bi: (bi, 0, 0)),
        ],
        out_shape=[
            jax.ShapeDtypeStruct((b, 1, _PAD), jnp.float32),
            jax.ShapeDtypeStruct((b, 1, _PAD), jnp.int32),
            jax.ShapeDtypeStruct((b, _TOPK, 16), jnp.float32),
            jax.ShapeDtypeStruct((b, _TOPK, 16), jnp.int32),
        ],
    )(e1r, e1i, pr, pi_, sr, si)

    del w16, d16
    return _tc_aggregate(wpad, dpad, values)


def _tc_aggregate(wpad, dpad, values):
    b, l, h, e = values.shape
    f = h * e
    fa = 128
    v2 = values.reshape(b, l, f)
    out = pl.pallas_call(
        _agg_kernel,
        grid=(b, f // fa),
        in_specs=[
            pl.BlockSpec(memory_space=pltpu.SMEM),
            pl.BlockSpec(memory_space=pltpu.SMEM),
            pl.BlockSpec((1, l, fa), lambda bi, fi: (bi, 0, fi)),
        ],
        out_specs=pl.BlockSpec((1, l, fa), lambda bi, fi: (bi, 0, fi)),
        out_shape=jax.ShapeDtypeStruct((b, l, f), jnp.float32),
        scratch_shapes=[pltpu.VMEM((2 * _L, fa), jnp.float32)],
    )(wpad, dpad, v2)
    return out.reshape(b, l, h, e)
